# baseline (speedup 1.0000x reference)
import functools

import jax
import jax.numpy as jnp
from jax import lax
from jax.experimental import pallas as pl
from jax.experimental.pallas import tpu as pltpu
from jax.experimental.pallas import tpu_sc as plsc

N_EMBED = 1024
SCALE = 32.0

_info = plsc.get_sparse_core_info()
_NC, _NS = _info.num_cores, _info.num_subcores
_NW = _NC * _NS


@functools.cache
def _make_gather(B, D):
    per_w = B // _NW
    K = 32
    n_chunks = per_w // K
    mesh = plsc.VectorSubcoreMesh(core_axis_name="c", subcore_axis_name="s")

    @functools.partial(
        pl.kernel,
        mesh=mesh,
        out_type=jax.ShapeDtypeStruct((B, D), jnp.float32),
        scratch_types=[
            pltpu.VMEM((per_w,), jnp.int32),
            pltpu.VMEM((3, K, D), jnp.float32),
            (pltpu.SemaphoreType.DMA,) * 3,
            (pltpu.SemaphoreType.DMA,) * 3,
        ],
    )
    def gather_kernel(table_hbm, idx_hbm, out_hbm, idx_v, rows_v, gsems, ssems):
        wid = lax.axis_index("s") * _NC + lax.axis_index("c")
        base = wid * per_w
        pltpu.sync_copy(idx_hbm.at[pl.ds(base, per_w)], idx_v)

        def G(i, b):
            return pltpu.make_async_copy(
                table_hbm.at[idx_v.at[pl.ds(i * K, K)]], rows_v.at[b], gsems[b])

        NH = 8
        H = K // NH

        def S_half(i, b, h):
            return pltpu.make_async_copy(
                rows_v.at[b].at[pl.ds(h * H, H)],
                out_hbm.at[pl.ds(base + i * K + h * H, H)], ssems[b])

        def scale_half(b, h):
            @plsc.parallel_loop(0, D // 16, unroll=2)
            def _col(j):
                sl = pl.ds(j * 16, 16)
                for r in range(h * H, (h + 1) * H):
                    rows_v[b, r, sl] = rows_v[b, r, sl] * SCALE

        def chunk(i, b, *, wait_prev_s=True, prefetch=True):
            if wait_prev_s:
                for h in range(NH):
                    S_half(i - 1, (b - 1) % 3, h).wait()
            if prefetch:
                G(i + 2, (b + 2) % 3).start()
            G(i, b).wait()
            for h in range(NH):
                scale_half(b, h)
                S_half(i, b, h).start()

        G(0, 0).start()
        G(1, 1).start()
        G(2, 2).start()
        chunk(0, 0, wait_prev_s=False, prefetch=False)

        @pl.loop(0, (n_chunks - 5) // 3)
        def _triple(j):
            c = 1 + 3 * j
            chunk(c, 1)
            chunk(c + 1, 2)
            chunk(c + 2, 0)

        chunk(n_chunks - 4, 1)
        chunk(n_chunks - 3, 2)
        chunk(n_chunks - 2, 0, prefetch=False)
        chunk(n_chunks - 1, 1, prefetch=False)
        for h in range(NH):
            S_half(n_chunks - 1, 1, h).wait()

    return gather_kernel


def kernel(x, table):
    B, S = x.shape
    _, D = table.shape
    idx = x.reshape(B * S).astype(jnp.int32)
    out = _make_gather(B * S, D)(table, idx)
    return out.reshape(B, S, D)


# speedup vs baseline: 1.0321x; 1.0321x over previous
import functools

import jax
import jax.numpy as jnp
from jax import lax
from jax.experimental import pallas as pl
from jax.experimental.pallas import tpu as pltpu
from jax.experimental.pallas import tpu_sc as plsc

N_EMBED = 1024
SCALE = 32.0

_info = plsc.get_sparse_core_info()
_NC, _NS = _info.num_cores, _info.num_subcores
_NW = _NC * _NS


@functools.cache
def _make_gather(B, D):
    per_w = B // _NW
    K = 32
    n_chunks = per_w // K
    mesh = plsc.VectorSubcoreMesh(core_axis_name="c", subcore_axis_name="s")

    @functools.partial(
        pl.kernel,
        mesh=mesh,
        out_type=jax.ShapeDtypeStruct((B, D), jnp.float32),
        scratch_types=[
            pltpu.VMEM((per_w,), jnp.int32),
            pltpu.VMEM((3, K, D), jnp.float32),
            (pltpu.SemaphoreType.DMA,) * 3,
            (pltpu.SemaphoreType.DMA,) * 3,
        ],
    )
    def gather_kernel(table_hbm, idx_hbm, out_hbm, idx_v, rows_v, gsems, ssems):
        wid = lax.axis_index("s") * _NC + lax.axis_index("c")
        base = wid * per_w
        pltpu.sync_copy(idx_hbm.at[pl.ds(base, per_w)], idx_v)

        def G(i, b):
            return pltpu.make_async_copy(
                table_hbm.at[idx_v.at[pl.ds(i * K, K)]], rows_v.at[b], gsems[b])

        NH = 4
        H = K // NH

        def S_half(i, b, h):
            return pltpu.make_async_copy(
                rows_v.at[b].at[pl.ds(h * H, H)],
                out_hbm.at[pl.ds(base + i * K + h * H, H)], ssems[b])

        def scale_half(b, h):
            @plsc.parallel_loop(0, D // 16, unroll=4)
            def _col(j):
                sl = pl.ds(j * 16, 16)
                for r in range(h * H, (h + 1) * H):
                    rows_v[b, r, sl] = rows_v[b, r, sl] * SCALE

        def chunk(i, b, *, wait_prev_s=True, prefetch=True):
            if wait_prev_s:
                for h in range(NH):
                    S_half(i - 1, (b - 1) % 3, h).wait()
            if prefetch:
                G(i + 2, (b + 2) % 3).start()
            G(i, b).wait()
            for h in range(NH):
                scale_half(b, h)
                S_half(i, b, h).start()

        G(0, 0).start()
        G(1, 1).start()
        G(2, 2).start()
        chunk(0, 0, wait_prev_s=False, prefetch=False)

        @pl.loop(0, (n_chunks - 5) // 3)
        def _triple(j):
            c = 1 + 3 * j
            chunk(c, 1)
            chunk(c + 1, 2)
            chunk(c + 2, 0)

        chunk(n_chunks - 4, 1)
        chunk(n_chunks - 3, 2)
        chunk(n_chunks - 2, 0, prefetch=False)
        chunk(n_chunks - 1, 1, prefetch=False)
        for h in range(NH):
            S_half(n_chunks - 1, 1, h).wait()

    return gather_kernel


def kernel(x, table):
    B, S = x.shape
    _, D = table.shape
    idx = x.reshape(B * S).astype(jnp.int32)
    out = _make_gather(B * S, D)(table, idx)
    return out.reshape(B, S, D)


# speedup vs baseline: 1.0354x; 1.0032x over previous
import functools

import jax
import jax.numpy as jnp
from jax import lax
from jax.experimental import pallas as pl
from jax.experimental.pallas import tpu as pltpu
from jax.experimental.pallas import tpu_sc as plsc

N_EMBED = 1024
SCALE = 32.0

_info = plsc.get_sparse_core_info()
_NC, _NS = _info.num_cores, _info.num_subcores
_NW = _NC * _NS


@functools.cache
def _make_gather(B, D):
    per_w = B // _NW
    K = 32
    n_chunks = per_w // K
    mesh = plsc.VectorSubcoreMesh(core_axis_name="c", subcore_axis_name="s")

    @functools.partial(
        pl.kernel,
        mesh=mesh,
        out_type=jax.ShapeDtypeStruct((B, D), jnp.float32),
        scratch_types=[
            pltpu.VMEM((per_w,), jnp.int32),
            pltpu.VMEM((3, K, D), jnp.float32),
            (pltpu.SemaphoreType.DMA,) * 3,
            (pltpu.SemaphoreType.DMA,) * 3,
        ],
    )
    def gather_kernel(table_hbm, idx_hbm, out_hbm, idx_v, rows_v, gsems, ssems):
        wid = lax.axis_index("s") * _NC + lax.axis_index("c")
        base = wid * per_w
        pltpu.sync_copy(idx_hbm.at[pl.ds(base, per_w)], idx_v)

        def G(i, b):
            return pltpu.make_async_copy(
                table_hbm.at[idx_v.at[pl.ds(i * K, K)]], rows_v.at[b], gsems[b])

        NH = 4
        H = K // NH

        def S_half(i, b, h):
            return pltpu.make_async_copy(
                rows_v.at[b].at[pl.ds(h * H, H)],
                out_hbm.at[pl.ds(base + i * K + h * H, H)], ssems[b])

        def scale_half(b, h):
            @plsc.parallel_loop(0, D // 16, unroll=2)
            def _col(j):
                sl = pl.ds(j * 16, 16)
                for r in range(h * H, (h + 1) * H):
                    rows_v[b, r, sl] = rows_v[b, r, sl] * SCALE

        def chunk(i, b, *, wait_prev_s=True, prefetch=True):
            if wait_prev_s:
                for h in range(NH):
                    S_half(i - 1, (b - 1) % 3, h).wait()
            if prefetch:
                G(i + 2, (b + 2) % 3).start()
            G(i, b).wait()
            for h in range(NH):
                scale_half(b, h)
                S_half(i, b, h).start()

        G(0, 0).start()
        G(1, 1).start()
        G(2, 2).start()
        chunk(0, 0, wait_prev_s=False, prefetch=False)

        @pl.loop(0, (n_chunks - 5) // 3)
        def _triple(j):
            c = 1 + 3 * j
            chunk(c, 1)
            chunk(c + 1, 2)
            chunk(c + 2, 0)

        chunk(n_chunks - 4, 1)
        chunk(n_chunks - 3, 2)
        chunk(n_chunks - 2, 0, prefetch=False)
        chunk(n_chunks - 1, 1, prefetch=False)
        for h in range(NH):
            S_half(n_chunks - 1, 1, h).wait()

    return gather_kernel


def kernel(x, table):
    B, S = x.shape
    _, D = table.shape
    idx = x.reshape(B * S).astype(jnp.int32)
    out = _make_gather(B * S, D)(table, idx)
    return out.reshape(B, S, D)


# speedup vs baseline: 1.0418x; 1.0062x over previous
import functools

import jax
import jax.numpy as jnp
from jax import lax
from jax.experimental import pallas as pl
from jax.experimental.pallas import tpu as pltpu
from jax.experimental.pallas import tpu_sc as plsc

N_EMBED = 1024
SCALE = 32.0

_info = plsc.get_sparse_core_info()
_NC, _NS = _info.num_cores, _info.num_subcores
_NW = _NC * _NS


@functools.cache
def _make_gather(B, D):
    per_w = B // _NW
    K = 32
    n_chunks = per_w // K
    mesh = plsc.VectorSubcoreMesh(core_axis_name="c", subcore_axis_name="s")

    @functools.partial(
        pl.kernel,
        mesh=mesh,
        out_type=jax.ShapeDtypeStruct((B, D), jnp.float32),
        scratch_types=[
            pltpu.VMEM((per_w,), jnp.int32),
            pltpu.VMEM((3, K, D), jnp.float32),
            (pltpu.SemaphoreType.DMA,) * 6,
            (pltpu.SemaphoreType.DMA,) * 3,
        ],
    )
    def gather_kernel(table_hbm, idx_hbm, out_hbm, idx_v, rows_v, gsems, ssems):
        wid = lax.axis_index("s") * _NC + lax.axis_index("c")
        base = wid * per_w
        pltpu.sync_copy(idx_hbm.at[pl.ds(base, per_w)], idx_v)

        def G(i, b, g):
            return pltpu.make_async_copy(
                table_hbm.at[idx_v.at[pl.ds(i * K + g * (K // 2), K // 2)]],
                rows_v.at[b].at[pl.ds(g * (K // 2), K // 2)], gsems[b * 2 + g])

        NH = 4
        H = K // NH

        def S_half(i, b, h):
            return pltpu.make_async_copy(
                rows_v.at[b].at[pl.ds(h * H, H)],
                out_hbm.at[pl.ds(base + i * K + h * H, H)], ssems[b])

        def scale_half(b, h):
            @plsc.parallel_loop(0, D // 16, unroll=2)
            def _col(j):
                sl = pl.ds(j * 16, 16)
                for r in range(h * H, (h + 1) * H):
                    rows_v[b, r, sl] = rows_v[b, r, sl] * SCALE

        def chunk(i, b, *, wait_prev_s=True, prefetch=True):
            if wait_prev_s:
                for h in range(NH):
                    S_half(i - 1, (b - 1) % 3, h).wait()
            if prefetch:
                G(i + 2, (b + 2) % 3, 0).start()
                G(i + 2, (b + 2) % 3, 1).start()
            for g in range(2):
                G(i, b, g).wait()
                for h in range(g * NH // 2, (g + 1) * NH // 2):
                    scale_half(b, h)
                    S_half(i, b, h).start()

        for c0 in range(3):
            G(c0, c0, 0).start()
            G(c0, c0, 1).start()
        chunk(0, 0, wait_prev_s=False, prefetch=False)

        @pl.loop(0, (n_chunks - 5) // 3)
        def _triple(j):
            c = 1 + 3 * j
            chunk(c, 1)
            chunk(c + 1, 2)
            chunk(c + 2, 0)

        chunk(n_chunks - 4, 1)
        chunk(n_chunks - 3, 2)
        chunk(n_chunks - 2, 0, prefetch=False)
        chunk(n_chunks - 1, 1, prefetch=False)
        for h in range(NH):
            S_half(n_chunks - 1, 1, h).wait()

    return gather_kernel


def kernel(x, table):
    B, S = x.shape
    _, D = table.shape
    idx = x.reshape(B * S).astype(jnp.int32)
    out = _make_gather(B * S, D)(table, idx)
    return out.reshape(B, S, D)
